# baseline (speedup 1.0000x reference)
import functools

import jax
import jax.numpy as jnp
from jax.experimental import pallas as pl
from jax.experimental.pallas import tpu as pltpu

KSEL = 256


_ERF_ALPHA = (-2.72614225801306e-10, 2.77068142495902e-08,
              -2.10102402082508e-06, -5.69250639462346e-05,
              -7.34990630326855e-04, -2.95459980854025e-03,
              -1.60960333262415e-02)
_ERF_BETA = (-1.45660718464996e-05, -2.13374055278905e-04,
             -1.68282697438203e-03, -7.37332916720468e-03,
             -1.42647390514189e-02)


def _erf(z):
    z = jnp.clip(z, -4.0, 4.0)
    z2 = z * z
    alpha = jnp.float32(_ERF_ALPHA[0])
    for c in _ERF_ALPHA[1:]:
        alpha = alpha * z2 + jnp.float32(c)
    beta = jnp.float32(_ERF_BETA[0])
    for c in _ERF_BETA[1:]:
        beta = beta * z2 + jnp.float32(c)
    return z * alpha / beta


def _gelu_exact(x):
    return 0.5 * x * (1.0 + _erf(x * jnp.float32(0.7071067811865476)))


def _gather_body(idx_ref, wt_ref, out_ref):
    del idx_ref
    out_ref[...] = wt_ref[...]


def _mm1_body(a_ref, w_ref, pa_ref, sc_ref):
    acts = jnp.dot(a_ref[0], w_ref[0], preferred_element_type=jnp.float32)
    pa = _gelu_exact(acts)
    pa_ref[0] = pa
    sc_ref[0] = jnp.sum(pa, axis=0, keepdims=True)


def _mask_body(sc_ref, mask_ref):
    s = sc_ref[0]
    n = s.shape[1]
    si = jax.lax.bitcast_convert_type(s, jnp.int32)
    keys = jnp.where(si >= 0, si, si ^ jnp.int32(0x7FFFFFFF))
    sign = jnp.int32(-2147483648)

    def tbody(i, p):
        bit = jnp.left_shift(jnp.int32(1), jnp.int32(31) - i)
        cand = p | bit
        cnt = jnp.sum(jnp.where(keys >= (cand ^ sign), jnp.int32(1), jnp.int32(0)))
        return jnp.where(cnt >= KSEL, cand, p)

    p = jax.lax.fori_loop(0, 32, tbody, jnp.int32(0))
    thr = p ^ sign

    gt = keys > thr
    eq = keys == thr
    deficit = KSEL - jnp.sum(jnp.where(gt, jnp.int32(1), jnp.int32(0)))
    pidx = jax.lax.broadcasted_iota(jnp.int32, (1, n), 1)

    def mbody(i, lohi):
        lo, hi = lohi
        mid = (lo + hi) // 2
        cnt = jnp.sum(jnp.where(eq & (pidx <= mid), jnp.int32(1), jnp.int32(0)))
        ok = cnt >= deficit
        return jnp.where(ok, lo, mid + 1), jnp.where(ok, mid, hi)

    lo, hi = jax.lax.fori_loop(0, 11, mbody, (jnp.int32(0), jnp.int32(n - 1)))
    mask = gt | (eq & (pidx <= lo))
    mask_ref[0] = mask.astype(jnp.float32)


def _mm2_body(pa_ref, mask_ref, p_ref, out_ref):
    pa = pa_ref[0] * mask_ref[0]
    out_ref[0] = jnp.dot(pa, p_ref[...], preferred_element_type=jnp.float32)


def kernel(selected_activations, selected_indices, k, combination_weights,
           output_projections):
    del k
    B, S, k_in = selected_activations.shape
    n_process, n_input = combination_weights.shape
    d_model = output_projections.shape[1]

    idx_flat = selected_indices.reshape(-1).astype(jnp.int32)
    w_t = combination_weights.T.reshape(n_input, 1, n_process)

    wsel = pl.pallas_call(
        _gather_body,
        grid_spec=pltpu.PrefetchScalarGridSpec(
            num_scalar_prefetch=1,
            grid=(B * k_in,),
            in_specs=[pl.BlockSpec((1, 1, n_process),
                                   lambda i, idx: (idx[i], 0, 0))],
            out_specs=pl.BlockSpec((1, 1, n_process), lambda i, idx: (i, 0, 0)),
        ),
        out_shape=jax.ShapeDtypeStruct((B * k_in, 1, n_process), jnp.float32),
    )(idx_flat, w_t)
    wsel = wsel.reshape(B, k_in, n_process)

    PBLK = 512
    pa, scores = pl.pallas_call(
        _mm1_body,
        grid=(B, n_process // PBLK),
        in_specs=[
            pl.BlockSpec((1, S, k_in), lambda b, p: (b, 0, 0)),
            pl.BlockSpec((1, k_in, PBLK), lambda b, p: (b, 0, p)),
        ],
        out_specs=[
            pl.BlockSpec((1, S, PBLK), lambda b, p: (b, 0, p)),
            pl.BlockSpec((1, 1, PBLK), lambda b, p: (b, 0, p)),
        ],
        out_shape=[
            jax.ShapeDtypeStruct((B, S, n_process), jnp.float32),
            jax.ShapeDtypeStruct((B, 1, n_process), jnp.float32),
        ],
    )(selected_activations, wsel)

    mask = pl.pallas_call(
        _mask_body,
        grid=(B,),
        in_specs=[pl.BlockSpec((1, 1, n_process), lambda b: (b, 0, 0))],
        out_specs=pl.BlockSpec((1, 1, n_process), lambda b: (b, 0, 0)),
        out_shape=jax.ShapeDtypeStruct((B, 1, n_process), jnp.float32),
    )(scores)

    SBLK = 512
    out = pl.pallas_call(
        _mm2_body,
        grid=(B, S // SBLK),
        in_specs=[
            pl.BlockSpec((1, SBLK, n_process), lambda b, s: (b, s, 0)),
            pl.BlockSpec((1, 1, n_process), lambda b, s: (b, 0, 0)),
            pl.BlockSpec((n_process, d_model), lambda b, s: (0, 0)),
        ],
        out_specs=pl.BlockSpec((1, SBLK, d_model), lambda b, s: (b, s, 0)),
        out_shape=jax.ShapeDtypeStruct((B, S, d_model), jnp.float32),
    )(pa, mask, output_projections)
    return out


# speedup vs baseline: 4.4729x; 4.4729x over previous
import functools

import jax
import jax.numpy as jnp
from jax import lax
from jax.experimental import pallas as pl
from jax.experimental.pallas import tpu as pltpu
from jax.experimental.pallas import tpu_sc as plsc

KSEL = 256

_SC_INFO = plsc.get_sparse_core_info()
_NC, _NS = _SC_INFO.num_cores, _SC_INFO.num_subcores
_NW = _NC * _NS


def _sc_gather_body(table_ref, idx_ref, out_ref, idx_v, rows_v, sem):
    n_rows = idx_v.shape[0]
    wid = lax.axis_index("s") * _NC + lax.axis_index("c")
    base = wid * n_rows
    pltpu.sync_copy(idx_ref.at[pl.ds(base, n_rows)], idx_v)
    pltpu.async_copy(table_ref.at[idx_v], rows_v, sem).wait()
    pltpu.sync_copy(rows_v, out_ref.at[pl.ds(base, n_rows)])


def _sc_gather_rows(table, idx):
    n_idx = idx.shape[0]
    d = table.shape[1]
    per_w = n_idx // _NW
    mesh = plsc.VectorSubcoreMesh(core_axis_name="c", subcore_axis_name="s")
    return pl.kernel(
        _sc_gather_body,
        out_type=jax.ShapeDtypeStruct((n_idx, d), table.dtype),
        mesh=mesh,
        scratch_types=[
            pltpu.VMEM((per_w,), jnp.int32),
            pltpu.VMEM((per_w, d), table.dtype),
            pltpu.SemaphoreType.DMA,
        ],
    )(table, idx)


_ERF_ALPHA = (-2.72614225801306e-10, 2.77068142495902e-08,
              -2.10102402082508e-06, -5.69250639462346e-05,
              -7.34990630326855e-04, -2.95459980854025e-03,
              -1.60960333262415e-02)
_ERF_BETA = (-1.45660718464996e-05, -2.13374055278905e-04,
             -1.68282697438203e-03, -7.37332916720468e-03,
             -1.42647390514189e-02)


def _erf(z):
    z = jnp.clip(z, -4.0, 4.0)
    z2 = z * z
    alpha = jnp.float32(_ERF_ALPHA[0])
    for c in _ERF_ALPHA[1:]:
        alpha = alpha * z2 + jnp.float32(c)
    beta = jnp.float32(_ERF_BETA[0])
    for c in _ERF_BETA[1:]:
        beta = beta * z2 + jnp.float32(c)
    return z * alpha / beta


def _gelu_exact(x):
    return 0.5 * x * (1.0 + _erf(x * jnp.float32(0.7071067811865476)))


def _gather_body(idx_ref, wt_ref, out_ref):
    del idx_ref
    out_ref[...] = wt_ref[...]


def _mm1_body(a_ref, w_ref, pa_ref, sc_ref):
    acts = jnp.dot(a_ref[0], w_ref[0], preferred_element_type=jnp.float32)
    pa = _gelu_exact(acts)
    pa_ref[0] = pa
    sc_ref[0] = jnp.sum(pa, axis=0, keepdims=True)


def _mask_body(sc_ref, mask_ref):
    s = sc_ref[0]
    n = s.shape[1]
    si = jax.lax.bitcast_convert_type(s, jnp.int32)
    keys = jnp.where(si >= 0, si, si ^ jnp.int32(0x7FFFFFFF))
    sign = jnp.int32(-2147483648)

    def tbody(i, p):
        bit = jnp.left_shift(jnp.int32(1), jnp.int32(31) - i)
        cand = p | bit
        cnt = jnp.sum(jnp.where(keys >= (cand ^ sign), jnp.int32(1), jnp.int32(0)))
        return jnp.where(cnt >= KSEL, cand, p)

    p = jax.lax.fori_loop(0, 32, tbody, jnp.int32(0))
    thr = p ^ sign

    gt = keys > thr
    eq = keys == thr
    deficit = KSEL - jnp.sum(jnp.where(gt, jnp.int32(1), jnp.int32(0)))
    pidx = jax.lax.broadcasted_iota(jnp.int32, (1, n), 1)

    def mbody(i, lohi):
        lo, hi = lohi
        mid = (lo + hi) // 2
        cnt = jnp.sum(jnp.where(eq & (pidx <= mid), jnp.int32(1), jnp.int32(0)))
        ok = cnt >= deficit
        return jnp.where(ok, lo, mid + 1), jnp.where(ok, mid, hi)

    lo, hi = jax.lax.fori_loop(0, 11, mbody, (jnp.int32(0), jnp.int32(n - 1)))
    mask = gt | (eq & (pidx <= lo))
    mask_ref[0] = mask.astype(jnp.float32)


def _mm2_body(pa_ref, mask_ref, p_ref, out_ref):
    pa = pa_ref[0] * mask_ref[0]
    out_ref[0] = jnp.dot(pa, p_ref[...], preferred_element_type=jnp.float32)


def kernel(selected_activations, selected_indices, k, combination_weights,
           output_projections):
    del k
    B, S, k_in = selected_activations.shape
    n_process, n_input = combination_weights.shape
    d_model = output_projections.shape[1]

    idx_flat = selected_indices.reshape(-1).astype(jnp.int32)
    w_t = combination_weights.T

    wsel = _sc_gather_rows(w_t, idx_flat)
    wsel = wsel.reshape(B, k_in, n_process)

    PBLK = 512
    pa, scores = pl.pallas_call(
        _mm1_body,
        grid=(B, n_process // PBLK),
        in_specs=[
            pl.BlockSpec((1, S, k_in), lambda b, p: (b, 0, 0)),
            pl.BlockSpec((1, k_in, PBLK), lambda b, p: (b, 0, p)),
        ],
        out_specs=[
            pl.BlockSpec((1, S, PBLK), lambda b, p: (b, 0, p)),
            pl.BlockSpec((1, 1, PBLK), lambda b, p: (b, 0, p)),
        ],
        out_shape=[
            jax.ShapeDtypeStruct((B, S, n_process), jnp.float32),
            jax.ShapeDtypeStruct((B, 1, n_process), jnp.float32),
        ],
    )(selected_activations, wsel)

    mask = pl.pallas_call(
        _mask_body,
        grid=(B,),
        in_specs=[pl.BlockSpec((1, 1, n_process), lambda b: (b, 0, 0))],
        out_specs=pl.BlockSpec((1, 1, n_process), lambda b: (b, 0, 0)),
        out_shape=jax.ShapeDtypeStruct((B, 1, n_process), jnp.float32),
    )(scores)

    SBLK = 512
    out = pl.pallas_call(
        _mm2_body,
        grid=(B, S // SBLK),
        in_specs=[
            pl.BlockSpec((1, SBLK, n_process), lambda b, s: (b, s, 0)),
            pl.BlockSpec((1, 1, n_process), lambda b, s: (b, 0, 0)),
            pl.BlockSpec((n_process, d_model), lambda b, s: (0, 0)),
        ],
        out_specs=pl.BlockSpec((1, SBLK, d_model), lambda b, s: (b, s, 0)),
        out_shape=jax.ShapeDtypeStruct((B, S, d_model), jnp.float32),
    )(pa, mask, output_projections)
    return out


# speedup vs baseline: 4.6780x; 1.0459x over previous
import functools

import jax
import jax.numpy as jnp
from jax import lax
from jax.experimental import pallas as pl
from jax.experimental.pallas import tpu as pltpu
from jax.experimental.pallas import tpu_sc as plsc

KSEL = 256

_NC, _NS = 2, 16
_NW = _NC * _NS

_ERF_ALPHA = (-2.72614225801306e-10, 2.77068142495902e-08,
              -2.10102402082508e-06, -5.69250639462346e-05,
              -7.34990630326855e-04, -2.95459980854025e-03,
              -1.60960333262415e-02)
_ERF_BETA = (-1.45660718464996e-05, -2.13374055278905e-04,
             -1.68282697438203e-03, -7.37332916720468e-03,
             -1.42647390514189e-02)


def _erf(z):
    z = jnp.clip(z, -4.0, 4.0)
    z2 = z * z
    alpha = jnp.float32(_ERF_ALPHA[0])
    for c in _ERF_ALPHA[1:]:
        alpha = alpha * z2 + jnp.float32(c)
    beta = jnp.float32(_ERF_BETA[0])
    for c in _ERF_BETA[1:]:
        beta = beta * z2 + jnp.float32(c)
    return z * alpha / beta


def _gelu_exact(x):
    return 0.5 * x * (1.0 + _erf(x * jnp.float32(0.7071067811865476)))


def _sc_gather_body(table_ref, idx_ref, out_ref, idx_v, rows_v, sem):
    n_rows = idx_v.shape[0]
    wid = lax.axis_index("s") * _NC + lax.axis_index("c")
    base = wid * n_rows
    pltpu.sync_copy(idx_ref.at[pl.ds(base, n_rows)], idx_v)
    pltpu.async_copy(table_ref.at[idx_v], rows_v, sem).wait()
    pltpu.sync_copy(rows_v, out_ref.at[pl.ds(base, n_rows)])


def _sc_gather_rows(table, idx):
    n_idx = idx.shape[0]
    d = table.shape[1]
    per_w = n_idx // _NW
    mesh = plsc.VectorSubcoreMesh(core_axis_name="c", subcore_axis_name="s")
    return pl.kernel(
        _sc_gather_body,
        out_type=jax.ShapeDtypeStruct((n_idx, d), table.dtype),
        mesh=mesh,
        scratch_types=[
            pltpu.VMEM((per_w,), jnp.int32),
            pltpu.VMEM((per_w, d), table.dtype),
            pltpu.SemaphoreType.DMA,
        ],
    )(table, idx)


def _topk_mask(s):
    n = s.shape[1]
    si = jax.lax.bitcast_convert_type(s, jnp.int32)
    keys = jnp.where(si >= 0, si, si ^ jnp.int32(0x7FFFFFFF))
    sign = jnp.int32(-2147483648)

    def tbody(i, p):
        bit = jnp.left_shift(jnp.int32(1), jnp.int32(31) - i)
        cand = p | bit
        cnt = jnp.sum(jnp.where(keys >= (cand ^ sign), jnp.int32(1), jnp.int32(0)))
        return jnp.where(cnt >= KSEL, cand, p)

    p = jax.lax.fori_loop(0, 32, tbody, jnp.int32(0))
    thr = p ^ sign

    gt = keys > thr
    eq = keys == thr
    deficit = KSEL - jnp.sum(jnp.where(gt, jnp.int32(1), jnp.int32(0)))
    pidx = jax.lax.broadcasted_iota(jnp.int32, (1, n), 1)

    def mbody(i, lohi):
        lo, hi = lohi
        mid = (lo + hi) // 2
        cnt = jnp.sum(jnp.where(eq & (pidx <= mid), jnp.int32(1), jnp.int32(0)))
        ok = cnt >= deficit
        return jnp.where(ok, lo, mid + 1), jnp.where(ok, mid, hi)

    lo, _ = jax.lax.fori_loop(0, 11, mbody, (jnp.int32(0), jnp.int32(n - 1)))
    return (gt | (eq & (pidx <= lo))).astype(jnp.float32)


_PT = 4
_ST = 4


def _fused_body(a_ref, w_ref, p_ref, out_ref, pa_scr, sc_scr, mask_scr):
    i = pl.program_id(1)
    s_full, pblk = pa_scr.shape[1], pa_scr.shape[2]
    sblk = out_ref.shape[1]

    @pl.when(i < _PT)
    def _mm1():
        acts = jnp.dot(a_ref[0], w_ref[0], preferred_element_type=jnp.float32)
        pa = _gelu_exact(acts)
        pa_scr[i] = pa
        sc_scr[i] = jnp.sum(pa, axis=0, keepdims=True)

    @pl.when(i == _PT)
    def _mask():
        s = jnp.concatenate([sc_scr[j] for j in range(_PT)], axis=-1)
        mask = _topk_mask(s)
        for j in range(_PT):
            mask_scr[j] = mask[:, j * pblk:(j + 1) * pblk]

    @pl.when(i >= _PT)
    def _mm2():
        st = i - _PT
        acc = jnp.zeros((sblk, out_ref.shape[2]), jnp.float32)
        for j in range(_PT):
            pa = pa_scr[j, pl.ds(st * sblk, sblk), :] * mask_scr[j]
            acc += jnp.dot(pa, p_ref[pl.ds(j * pblk, pblk), :],
                           preferred_element_type=jnp.float32)
        out_ref[0] = acc


def kernel(selected_activations, selected_indices, k, combination_weights,
           output_projections):
    del k
    B, S, k_in = selected_activations.shape
    n_process, n_input = combination_weights.shape
    d_model = output_projections.shape[1]
    PBLK = n_process // _PT
    SBLK = S // _ST

    idx_flat = selected_indices.reshape(-1).astype(jnp.int32)
    w_t = combination_weights.T

    wsel = _sc_gather_rows(w_t, idx_flat).reshape(B, k_in, n_process)

    out = pl.pallas_call(
        _fused_body,
        grid=(B, _PT + _ST),
        in_specs=[
            pl.BlockSpec((1, S, k_in), lambda b, i: (b, 0, 0)),
            pl.BlockSpec((1, k_in, PBLK),
                         lambda b, i: (b, 0, jnp.minimum(i, _PT - 1))),
            pl.BlockSpec((n_process, d_model), lambda b, i: (0, 0)),
        ],
        out_specs=pl.BlockSpec(
            (1, SBLK, d_model),
            lambda b, i: (b, jnp.where(i < _PT, 0, i - _PT), 0)),
        out_shape=jax.ShapeDtypeStruct((B, S, d_model), jnp.float32),
        scratch_shapes=[
            pltpu.VMEM((_PT, S, PBLK), jnp.float32),
            pltpu.VMEM((_PT, 1, PBLK), jnp.float32),
            pltpu.VMEM((_PT, 1, PBLK), jnp.float32),
        ],
    )(selected_activations, wsel, output_projections)
    return out
